# speedup vs baseline: 21.7214x; 21.7214x over previous
import jax
import jax.numpy as jnp
from jax.experimental import pallas as pl
from jax.experimental.pallas import tpu as pltpu

_KP1 = 33
_ROW_BLOCK = 256


def _emb_kernel(f_ref, w1_ref, b1_ref, w2_ref, b2_ref, out_ref):
    f = f_ref[...]
    h = jax.lax.dot_general(f, w1_ref[...], (((1,), (1,)), ((), ())),
                            preferred_element_type=jnp.float32)
    h = jnp.maximum(h + b1_ref[...], 0.0)
    e = jax.lax.dot_general(h, w2_ref[...], (((1,), (1,)), ((), ())),
                            preferred_element_type=jnp.float32)
    e = e + b2_ref[...]
    norm = jnp.sqrt(jnp.sum(e * e, axis=1, keepdims=True))
    out_ref[...] = e / jnp.maximum(norm, 1e-12)


def _sim_kernel(eb_ref, ef_ref, out_ref):
    i = pl.program_id(0)
    eb = eb_ref[...]
    ef = ef_ref[...]
    sim = jax.lax.dot_general(eb, ef, (((1,), (1,)), ((), ())),
                              preferred_element_type=jnp.float32)
    br, n = sim.shape

    b = jax.lax.bitcast_convert_type(sim, jnp.int32)
    m = jnp.where(b >= 0, b, jnp.int32(-2147483648) - b)

    lo0 = jnp.full((br, 1), -1073741825, jnp.int32)
    hi0 = jnp.full((br, 1), 1073741825, jnp.int32)

    def body(_, carry):
        lo, hi = carry
        mid = (lo + hi) >> 1
        cnt = jnp.sum((m >= mid).astype(jnp.float32), axis=1, keepdims=True)
        ge = cnt >= float(_KP1)
        return jnp.where(ge, mid, lo), jnp.where(ge, hi, mid)

    lo, _ = jax.lax.fori_loop(0, 31, body, (lo0, hi0))

    col = jax.lax.broadcasted_iota(jnp.int32, (br, n), 1)
    row = i * br + jax.lax.broadcasted_iota(jnp.int32, (br, n), 0)
    keep = (m >= lo) & (sim > 0.0) & (col != row)
    out_ref[...] = jnp.where(keep, sim, 0.0)


def kernel(features, W1, b1, W2, b2):
    n, d = features.shape
    emb = pl.pallas_call(
        _emb_kernel,
        out_shape=jax.ShapeDtypeStruct((n, d), jnp.float32),
    )(features, W1, b1.reshape(1, d), W2, b2.reshape(1, d))

    grid = n // _ROW_BLOCK
    out = pl.pallas_call(
        _sim_kernel,
        grid=(grid,),
        in_specs=[
            pl.BlockSpec((_ROW_BLOCK, d), lambda i: (i, 0)),
            pl.BlockSpec((n, d), lambda i: (0, 0)),
        ],
        out_specs=pl.BlockSpec((_ROW_BLOCK, n), lambda i: (i, 0)),
        out_shape=jax.ShapeDtypeStruct((n, n), jnp.float32),
    )(emb, emb)
    return out


# speedup vs baseline: 24.3115x; 1.1192x over previous
import jax
import jax.numpy as jnp
from jax.experimental import pallas as pl
from jax.experimental.pallas import tpu as pltpu

_KP1 = 33
_ROW_BLOCK = 256


def _emb_kernel(f_ref, w1_ref, b1_ref, w2_ref, b2_ref, out_ref):
    f = f_ref[...]
    h = jax.lax.dot_general(f, w1_ref[...], (((1,), (1,)), ((), ())),
                            preferred_element_type=jnp.float32)
    h = jnp.maximum(h + b1_ref[...], 0.0)
    e = jax.lax.dot_general(h, w2_ref[...], (((1,), (1,)), ((), ())),
                            preferred_element_type=jnp.float32)
    e = e + b2_ref[...]
    norm = jnp.sqrt(jnp.sum(e * e, axis=1, keepdims=True))
    out_ref[...] = e / jnp.maximum(norm, 1e-12)


def _sim_kernel(eb_ref, ef_ref, out_ref):
    i = pl.program_id(0)
    eb = eb_ref[...]
    ef = ef_ref[...]
    sim = jax.lax.dot_general(eb, ef, (((1,), (1,)), ((), ())),
                              preferred_element_type=jnp.float32)
    br, n = sim.shape

    def _map(v):
        bb = jax.lax.bitcast_convert_type(v, jnp.int32)
        return jnp.where(bb >= 0, bb, jnp.int32(-2147483648) - bb)

    def _unmap(mi):
        bb = jnp.where(mi >= 0, mi, jnp.int32(-2147483648) - mi)
        return jax.lax.bitcast_convert_type(bb, jnp.float32)

    low_i32 = jnp.int32(-1073741824)
    rmax = jnp.max(sim, axis=1, keepdims=True)
    hi0 = jnp.minimum(_map(rmax), jnp.int32(1073741822)) + 1
    lo0 = jnp.full((br, 1), low_i32, jnp.int32)

    ssub = sim[:, :512]

    def pre_body(_, carry):
        lo, hi = carry
        mid = lo + ((hi - lo) >> 1)
        cnt = jnp.sum((ssub >= _unmap(mid)).astype(jnp.float32), axis=1,
                      keepdims=True)
        ge = cnt >= float(_KP1)
        return jnp.where(ge, mid, lo), jnp.where(ge, hi, mid)

    lo1, _ = jax.lax.fori_loop(0, 20, pre_body, (lo0, hi0))

    def cond(carry):
        lo, hi = carry
        return jnp.max(hi - lo) > 1

    def body(carry):
        lo, hi = carry
        mid = lo + ((hi - lo) >> 1)
        cnt = jnp.sum((sim >= _unmap(mid)).astype(jnp.float32), axis=1,
                      keepdims=True)
        ge = cnt >= float(_KP1)
        eq = cnt == float(_KP1)
        lo = jnp.where(ge, mid, lo)
        hi = jnp.where(eq, mid + 1, jnp.where(ge, hi, mid))
        return lo, hi

    lo, _ = jax.lax.while_loop(cond, body, (lo1, hi0))

    thr = _unmap(jnp.maximum(lo, 1))
    out_ref[...] = jnp.where(sim >= thr, sim, 0.0)

    eye = (jax.lax.broadcasted_iota(jnp.int32, (br, br), 0)
           == jax.lax.broadcasted_iota(jnp.int32, (br, br), 1))
    dsub = out_ref[:, pl.ds(i * br, br)]
    out_ref[:, pl.ds(i * br, br)] = jnp.where(eye, 0.0, dsub)


def kernel(features, W1, b1, W2, b2):
    n, d = features.shape
    emb = pl.pallas_call(
        _emb_kernel,
        out_shape=jax.ShapeDtypeStruct((n, d), jnp.float32),
    )(features, W1, b1.reshape(1, d), W2, b2.reshape(1, d))

    grid = n // _ROW_BLOCK
    out = pl.pallas_call(
        _sim_kernel,
        grid=(grid,),
        in_specs=[
            pl.BlockSpec((_ROW_BLOCK, d), lambda i: (i, 0)),
            pl.BlockSpec((n, d), lambda i: (0, 0)),
        ],
        out_specs=pl.BlockSpec((_ROW_BLOCK, n), lambda i: (i, 0)),
        out_shape=jax.ShapeDtypeStruct((n, n), jnp.float32),
    )(emb, emb)
    return out
